# baseline (speedup 1.0000x reference)
import functools

import jax
import jax.numpy as jnp
from jax import lax
from jax.experimental import pallas as pl
from jax.experimental.pallas import tpu as pltpu
from jax.experimental.pallas import tpu_sc as plsc


def kernel(items, emb_bag_inputs, offsets, input_lengths, emb_table):
    n_items = offsets.shape[0]
    L = emb_bag_inputs.shape[0] // n_items
    B = items.shape[0]
    V, D = emb_table.shape

    info = plsc.get_sparse_core_info()
    LN = info.num_lanes
    NC = info.num_cores
    NS = info.num_subcores
    b_per_t = B // NS
    SUB = 512
    E = SUB * L
    CH = 512
    n_ch = B // CH

    mesh = plsc.VectorSubcoreMesh(core_axis_name="c", subcore_axis_name="s")

    @functools.partial(
        pl.kernel,
        mesh=mesh,
        out_type=(jax.ShapeDtypeStruct((D, B), jnp.float32),
                  jax.ShapeDtypeStruct((NC, L, B), jnp.int32)),
        compiler_params=pltpu.CompilerParams(
            needs_layout_passes=False, use_tc_tiling_on_sc=False),
        scratch_types=[
            pltpu.VMEM((SUB,), jnp.int32),
            pltpu.VMEM((E,), jnp.int32),
            pltpu.VMEM((E,), jnp.int32),
            pltpu.VMEM((CH * L,), jnp.int32),
            pltpu.VMEM((V,), jnp.float32),
            pltpu.VMEM((B,), jnp.float32),
            pltpu.SemaphoreType.DMA,
            pltpu.SemaphoreType.DMA,
        ],
    )
    def bag_kernel(items_hbm, bag_hbm, tablet_hbm, outt_hbm, featst_hbm,
                   items_v, eidx_v, feat_v, featc_v, trow_v, ocol_v,
                   sem_row, sem_gat):
        cid = lax.axis_index("c")
        sid = lax.axis_index("s")
        d = sid * NC + cid

        row_cp = pltpu.async_copy(tablet_hbm.at[d], trow_v, sem_row)

        lane = lax.iota(jnp.int32, LN)
        for sub in range(b_per_t // SUB):
            ibase = sid * b_per_t + sub * SUB
            pltpu.sync_copy(items_hbm.at[pl.ds(ibase, SUB)], items_v)

            @plsc.parallel_loop(0, SUB // LN, 1, unroll=2)
            def expand_body(g):
                iv = items_v[pl.ds(g * LN, LN)] * L
                for j in range(L):
                    eidx_v[pl.ds(j * SUB + g * LN, LN)] = iv + j

            pltpu.async_copy(bag_hbm.at[eidx_v], feat_v, sem_gat).wait()
            for j in range(L):
                pltpu.sync_copy(
                    feat_v.at[pl.ds(j * SUB, SUB)],
                    featst_hbm.at[cid, j, pl.ds(ibase, SUB)])

        plsc.subcore_barrier()
        row_cp.wait()

        def chunk_body(ch, carry):
            cbase = ch * CH
            for j in range(L):
                pltpu.sync_copy(
                    featst_hbm.at[cid, j, pl.ds(cbase, CH)],
                    featc_v.at[pl.ds(j * CH, CH)])

            @plsc.parallel_loop(0, CH // LN, 1, unroll=2)
            def group_body(g):
                acc = None
                for j in range(L):
                    fids = featc_v[pl.ds(j * CH + g * LN, LN)]
                    vals = plsc.load_gather(trow_v, [fids])
                    acc = vals if acc is None else acc + vals
                ocol_v[pl.ds(cbase + g * LN, LN)] = acc

            return carry

        lax.fori_loop(0, n_ch, chunk_body, 0)
        pltpu.sync_copy(ocol_v, outt_hbm.at[d])

    out_t, _ = bag_kernel(items, emb_bag_inputs, emb_table.T)
    return out_t.T


# speedup vs baseline: 2.4618x; 2.4618x over previous
import functools

import jax
import jax.numpy as jnp
from jax import lax
from jax.experimental import pallas as pl
from jax.experimental.pallas import tpu as pltpu
from jax.experimental.pallas import tpu_sc as plsc


def kernel(items, emb_bag_inputs, offsets, input_lengths, emb_table):
    n_items = offsets.shape[0]
    L = emb_bag_inputs.shape[0] // n_items
    B = items.shape[0]
    V, D = emb_table.shape

    info = plsc.get_sparse_core_info()
    LN = info.num_lanes
    NC = info.num_cores
    NS = info.num_subcores
    b_per_t = B // NS
    SUB = 512
    E = SUB * L
    CH = 512
    n_ch = B // CH

    mesh = plsc.VectorSubcoreMesh(core_axis_name="c", subcore_axis_name="s")

    @functools.partial(
        pl.kernel,
        mesh=mesh,
        out_type=(jax.ShapeDtypeStruct((D, B), jnp.float32),
                  jax.ShapeDtypeStruct((n_ch, E), jnp.int32)),
        compiler_params=pltpu.CompilerParams(
            needs_layout_passes=False, use_tc_tiling_on_sc=False),
        scratch_types=[
            pltpu.VMEM((SUB,), jnp.int32),
            pltpu.VMEM((E,), jnp.int32),
            pltpu.VMEM((E,), jnp.int32),
            pltpu.VMEM((E,), jnp.int32),
            pltpu.VMEM((E,), jnp.int32),
            pltpu.VMEM((V,), jnp.float32),
            pltpu.VMEM((B // 2,), jnp.float32),
            pltpu.SemaphoreType.DMA,
            pltpu.SemaphoreType.DMA,
            pltpu.SemaphoreType.DMA,
        ],
    )
    def bag_kernel(items_hbm, bag_hbm, tablet_hbm, outt_hbm, featst_hbm,
                   items_v, eidx_v, feat_v, featc0_v, featc1_v, trow_v,
                   ocol_v, sem_row, sem0, sem1):
        cid = lax.axis_index("c")
        sid = lax.axis_index("s")
        d = sid * NC + cid

        row_cp = pltpu.async_copy(tablet_hbm.at[d], trow_v, sem_row)

        for sub in range(b_per_t // SUB):
            ibase = sid * b_per_t + sub * SUB
            pltpu.sync_copy(items_hbm.at[pl.ds(ibase, SUB)], items_v)

            @plsc.parallel_loop(0, SUB // LN, 1, unroll=2)
            def expand_body(g):
                iv = items_v[pl.ds(g * LN, LN)] * L
                for j in range(L):
                    eidx_v[pl.ds(j * SUB + g * LN, LN)] = iv + j

            pltpu.async_copy(bag_hbm.at[eidx_v], feat_v, sem0).wait()
            pltpu.sync_copy(feat_v, featst_hbm.at[sid * 2 + sub])

        plsc.subcore_barrier()
        row_cp.wait()

        bufs = (featc0_v, featc1_v)
        sems = (sem0, sem1)

        def start(ch):
            return pltpu.async_copy(featst_hbm.at[ch], bufs[ch % 2],
                                    sems[ch % 2])

        pending = start(0)
        half = n_ch // 2
        for ch in range(n_ch):
            nxt = start(ch + 1) if ch + 1 < n_ch else None
            pending.wait()
            pending = nxt
            featc_v = bufs[ch % 2]
            cbase = (ch % half) * CH

            @plsc.parallel_loop(0, CH // LN, 1, unroll=2)
            def group_body(g):
                acc = None
                for j in range(L):
                    fids = featc_v[pl.ds(j * CH + g * LN, LN)]
                    vals = plsc.load_gather(trow_v, [fids])
                    acc = vals if acc is None else acc + vals
                ocol_v[pl.ds(cbase + g * LN, LN)] = acc

            if ch == half - 1:
                pltpu.sync_copy(ocol_v, outt_hbm.at[d, pl.ds(0, B // 2)])
        pltpu.sync_copy(ocol_v, outt_hbm.at[d, pl.ds(B // 2, B // 2)])

    out_t, _ = bag_kernel(items, emb_bag_inputs, emb_table.T)
    return out_t.T


# speedup vs baseline: 2.4638x; 1.0008x over previous
import functools

import jax
import jax.numpy as jnp
from jax import lax
from jax.experimental import pallas as pl
from jax.experimental.pallas import tpu as pltpu
from jax.experimental.pallas import tpu_sc as plsc


def kernel(items, emb_bag_inputs, offsets, input_lengths, emb_table):
    n_items = offsets.shape[0]
    L = emb_bag_inputs.shape[0] // n_items
    B = items.shape[0]
    V, D = emb_table.shape

    info = plsc.get_sparse_core_info()
    LN = info.num_lanes
    NC = info.num_cores
    NS = info.num_subcores
    b_per_t = B // NS
    SUB = 512
    E = SUB * L
    CH = 512
    n_ch = B // CH

    mesh = plsc.VectorSubcoreMesh(core_axis_name="c", subcore_axis_name="s")

    @functools.partial(
        pl.kernel,
        mesh=mesh,
        out_type=(jax.ShapeDtypeStruct((D, B), jnp.float32),
                  jax.ShapeDtypeStruct((n_ch, E), jnp.int32)),
        compiler_params=pltpu.CompilerParams(
            needs_layout_passes=False, use_tc_tiling_on_sc=False),
        scratch_types=[
            pltpu.VMEM((SUB,), jnp.int32),
            pltpu.VMEM((E,), jnp.int32),
            pltpu.VMEM((E,), jnp.int32),
            pltpu.VMEM((E,), jnp.int32),
            pltpu.VMEM((E,), jnp.int32),
            pltpu.VMEM((V,), jnp.float32),
            pltpu.VMEM((B // 2,), jnp.float32),
            pltpu.SemaphoreType.DMA,
            pltpu.SemaphoreType.DMA,
            pltpu.SemaphoreType.DMA,
        ],
    )
    def bag_kernel(items_hbm, bag_hbm, tablet_hbm, outt_hbm, featst_hbm,
                   items_v, eidx_v, feat_v, featc0_v, featc1_v, trow_v,
                   ocol_v, sem_row, sem0, sem1):
        cid = lax.axis_index("c")
        sid = lax.axis_index("s")
        d = sid * NC + cid

        row_cp = pltpu.async_copy(tablet_hbm.at[d], trow_v, sem_row)

        def expand(sub, dst):
            ibase = sid * b_per_t + sub * SUB
            pltpu.sync_copy(items_hbm.at[pl.ds(ibase, SUB)], items_v)

            @plsc.parallel_loop(0, SUB // LN, 1, unroll=2)
            def expand_body(g):
                iv = items_v[pl.ds(g * LN, LN)] * L
                for j in range(L):
                    dst[pl.ds(j * SUB + g * LN, LN)] = iv + j

        expand(0, eidx_v)
        gat0 = pltpu.async_copy(bag_hbm.at[eidx_v], feat_v, sem0)
        expand(1, featc0_v)
        gat1 = pltpu.async_copy(bag_hbm.at[featc0_v], featc1_v, sem1)
        gat0.wait()
        st0 = pltpu.async_copy(feat_v, featst_hbm.at[sid * 2], sem0)
        gat1.wait()
        st1 = pltpu.async_copy(featc1_v, featst_hbm.at[sid * 2 + 1], sem1)
        st0.wait()
        st1.wait()

        plsc.subcore_barrier()
        row_cp.wait()

        bufs = (featc0_v, featc1_v)
        sems = (sem0, sem1)

        def start(ch):
            return pltpu.async_copy(featst_hbm.at[ch], bufs[ch % 2],
                                    sems[ch % 2])

        pending = start(0)
        half = n_ch // 2
        for ch in range(n_ch):
            nxt = start(ch + 1) if ch + 1 < n_ch else None
            pending.wait()
            pending = nxt
            featc_v = bufs[ch % 2]
            cbase = (ch % half) * CH

            @plsc.parallel_loop(0, CH // LN, 1, unroll=4)
            def group_body(g):
                acc = None
                for j in range(L):
                    fids = featc_v[pl.ds(j * CH + g * LN, LN)]
                    vals = plsc.load_gather(trow_v, [fids])
                    acc = vals if acc is None else acc + vals
                ocol_v[pl.ds(cbase + g * LN, LN)] = acc

            if ch == half - 1:
                pltpu.sync_copy(ocol_v, outt_hbm.at[d, pl.ds(0, B // 2)])
        pltpu.sync_copy(ocol_v, outt_hbm.at[d, pl.ds(B // 2, B // 2)])

    out_t, _ = bag_kernel(items, emb_bag_inputs, emb_table.T)
    return out_t.T


# speedup vs baseline: 2.9111x; 1.1815x over previous
import functools

import jax
import jax.numpy as jnp
from jax import lax
from jax.experimental import pallas as pl
from jax.experimental.pallas import tpu as pltpu
from jax.experimental.pallas import tpu_sc as plsc


def kernel(items, emb_bag_inputs, offsets, input_lengths, emb_table):
    n_items = offsets.shape[0]
    L = emb_bag_inputs.shape[0] // n_items
    B = items.shape[0]
    V, D = emb_table.shape

    info = plsc.get_sparse_core_info()
    LN = info.num_lanes
    NC = info.num_cores
    NS = info.num_subcores
    b_per_t = B // NS
    SUB = 512
    E = SUB * L
    CH = 512
    n_ch = B // CH

    mesh = plsc.VectorSubcoreMesh(core_axis_name="c", subcore_axis_name="s")

    @functools.partial(
        pl.kernel,
        mesh=mesh,
        out_type=jax.ShapeDtypeStruct((D, B), jnp.float32),
        compiler_params=pltpu.CompilerParams(
            needs_layout_passes=False, use_tc_tiling_on_sc=False),
        scratch_types=[
            pltpu.VMEM((SUB,), jnp.int32),
            pltpu.VMEM((E,), jnp.int32),
            pltpu.VMEM((E,), jnp.int32),
            pltpu.VMEM((E,), jnp.int32),
            pltpu.VMEM((E,), jnp.int32),
            pltpu.VMEM((V,), jnp.float32),
            pltpu.VMEM((B // 4,), jnp.float32),
            pltpu.VMEM_SHARED((n_ch, E), jnp.int32),
            pltpu.SemaphoreType.DMA,
            pltpu.SemaphoreType.DMA,
            pltpu.SemaphoreType.DMA,
        ],
    )
    def bag_kernel(items_hbm, bag_hbm, tablet_hbm, outt_hbm,
                   items_v, eidx_v, feat_v, featc0_v, featc1_v, trow_v,
                   ocol_v, sh_feat, sem_row, sem0, sem1):
        cid = lax.axis_index("c")
        sid = lax.axis_index("s")
        d = sid * NC + cid

        row_cp = pltpu.async_copy(tablet_hbm.at[d], trow_v, sem_row)

        def expand(sub, dst):
            ibase = sid * b_per_t + sub * SUB
            pltpu.sync_copy(items_hbm.at[pl.ds(ibase, SUB)], items_v)

            @plsc.parallel_loop(0, SUB // LN, 1, unroll=2)
            def expand_body(g):
                iv = items_v[pl.ds(g * LN, LN)] * L
                for j in range(L):
                    dst[pl.ds(j * SUB + g * LN, LN)] = iv + j

        expand(0, eidx_v)
        gat0 = pltpu.async_copy(bag_hbm.at[eidx_v], feat_v, sem0)
        expand(1, featc0_v)
        gat1 = pltpu.async_copy(bag_hbm.at[featc0_v], featc1_v, sem1)
        gat0.wait()
        st0 = pltpu.async_copy(feat_v, sh_feat.at[sid * 2], sem0)
        gat1.wait()
        st1 = pltpu.async_copy(featc1_v, sh_feat.at[sid * 2 + 1], sem1)
        st0.wait()
        st1.wait()

        plsc.subcore_barrier()
        row_cp.wait()

        bufs = (featc0_v, featc1_v)
        sems = (sem0, sem1)

        def start(ch):
            return pltpu.async_copy(sh_feat.at[ch], bufs[ch % 2],
                                    sems[ch % 2])

        quarter = n_ch // 4
        pending = start(0)
        for ch in range(n_ch):
            nxt = start(ch + 1) if ch + 1 < n_ch else None
            pending.wait()
            pending = nxt
            featc_v = bufs[ch % 2]
            cbase = (ch % quarter) * CH

            @plsc.parallel_loop(0, CH // LN, 1, unroll=4)
            def group_body(g):
                acc = None
                for j in range(L):
                    fids = featc_v[pl.ds(j * CH + g * LN, LN)]
                    vals = plsc.load_gather(trow_v, [fids])
                    acc = vals if acc is None else acc + vals
                ocol_v[pl.ds(cbase + g * LN, LN)] = acc

            if ch % quarter == quarter - 1:
                pltpu.sync_copy(
                    ocol_v,
                    outt_hbm.at[d, pl.ds((ch // quarter) * (B // 4), B // 4)])

    out_t = bag_kernel(items, emb_bag_inputs, emb_table.T)
    return out_t.T
